# baseline (speedup 1.0000x reference)
import functools

import jax
import jax.numpy as jnp
from jax import lax
from jax.experimental import pallas as pl
from jax.experimental.pallas import tpu as pltpu
from jax.experimental.pallas import tpu_sc as plsc

_NUM_CORES = 2
_NUM_SUBCORES = 16
_NUM_WORKERS = _NUM_CORES * _NUM_SUBCORES
_CHUNK = 32
_NBUF = 4


def _tc_pack(x, rows_per_block):
    R, D = x.shape
    Dh = D // 2

    def body(x_ref, out_ref):
        u = lax.bitcast_convert_type(x_ref[...], jnp.uint32)
        au = u[:, :Dh] + 0x8000
        bu = u[:, Dh:] + 0x8000
        out_ref[...] = lax.bitcast_convert_type(
            (au >> 16) | (bu & jnp.uint32(0xFFFF0000)), jnp.int32
        )

    return pl.pallas_call(
        body,
        grid=(R // rows_per_block,),
        in_specs=[pl.BlockSpec((rows_per_block, D), lambda i: (i, 0))],
        out_specs=pl.BlockSpec((rows_per_block, Dh), lambda i: (i, 0)),
        out_shape=jax.ShapeDtypeStruct((R, Dh), jnp.int32),
    )(x)


def _sc_gather(x_words, idx):
    V, Dw = x_words.shape
    _, n_chunks, _ = idx.shape
    rows_per_w = n_chunks * _CHUNK
    total_rows = _NUM_WORKERS * rows_per_w
    n_groups = n_chunks // _NBUF

    mesh = plsc.VectorSubcoreMesh(
        core_axis_name="c",
        subcore_axis_name="s",
        num_cores=_NUM_CORES,
        num_subcores=_NUM_SUBCORES,
    )

    @functools.partial(
        pl.kernel,
        mesh=mesh,
        out_type=jax.ShapeDtypeStruct((total_rows, Dw), jnp.int32),
        scratch_types=[
            pltpu.VMEM((n_chunks, _CHUNK), jnp.int32),
            pltpu.VMEM((_NBUF, _CHUNK, Dw), jnp.int32),
            pltpu.SemaphoreType.DMA,
            pltpu.SemaphoreType.DMA,
        ],
    )
    def gather_kernel(idx_hbm, x_hbm, out_hbm, idx_v, bufs, gsem, ssem):
        wid = lax.axis_index("s") * _NUM_CORES + lax.axis_index("c")
        base = wid * rows_per_w
        pltpu.sync_copy(idx_hbm.at[wid], idx_v)

        def gather(c, i):
            return pltpu.make_async_copy(x_hbm.at[idx_v.at[c]], bufs.at[i], gsem)

        def store(c, i):
            return pltpu.make_async_copy(
                bufs.at[i], out_hbm.at[pl.ds(base + c * _CHUNK, _CHUNK)], ssem
            )

        for i in range(_NBUF):
            gather(i, i).start()

        def body(g, _):
            c0 = _NBUF * g
            for i in range(_NBUF):
                gather(c0 + i, i).wait()
                store(c0 + i, i).start()
            for i in range(_NBUF):
                store(c0 + i, i).wait()

                @pl.when(c0 + i + _NBUF < n_chunks)
                def _start_next(c=c0 + i + _NBUF, i=i):
                    gather(c, i).start()

            return None

        lax.fori_loop(0, n_groups, body, None)

    return gather_kernel(idx, x_words)


def _tc_cast_w(w):
    H, D, HD = w.shape

    def body(w_ref, out_ref):
        out_ref[...] = w_ref[...].astype(jnp.bfloat16)

    return pl.pallas_call(
        body,
        grid=(H,),
        in_specs=[pl.BlockSpec((1, D, HD), lambda h: (h, 0, 0))],
        out_specs=pl.BlockSpec((1, D, HD), lambda h: (h, 0, 0)),
        out_shape=jax.ShapeDtypeStruct((H, D, HD), jnp.bfloat16),
    )(w)


def _tc_matmul(xg_words, w):
    B, H, K, Dw = xg_words.shape
    D = 2 * Dw
    HD = w.shape[2]

    HPB = 8

    def body(xg_ref, w_ref, out_ref):
        dn = (((1,), (0,)), ((), ()))
        for i in range(HPB):
            words = lax.bitcast_convert_type(xg_ref[0, i], jnp.uint32)
            lo = lax.bitcast_convert_type(words << 16, jnp.float32).astype(
                jnp.bfloat16
            )
            hi = lax.bitcast_convert_type(
                words & jnp.uint32(0xFFFF0000), jnp.float32
            ).astype(jnp.bfloat16)
            out_ref[0, i] = lax.dot_general(
                lo, w_ref[i, :Dw], dn, preferred_element_type=jnp.float32
            ) + lax.dot_general(
                hi, w_ref[i, Dw:], dn, preferred_element_type=jnp.float32
            )

    return pl.pallas_call(
        body,
        grid=(H // HPB, B),
        in_specs=[
            pl.BlockSpec((1, HPB, K, Dw), lambda h, b: (b, h, 0, 0)),
            pl.BlockSpec((HPB, D, HD), lambda h, b: (h, 0, 0)),
        ],
        out_specs=pl.BlockSpec((1, HPB, K, HD), lambda h, b: (b, h, 0, 0)),
        out_shape=jax.ShapeDtypeStruct((B, H, K, HD), jnp.float32),
    )(xg_words, w)


def kernel(X, ind, W):
    B, N, D = X.shape
    _, H, K = ind.shape

    total_rows = B * H * K
    rows_per_w = total_rows // _NUM_WORKERS
    n_chunks = rows_per_w // _CHUNK

    idx = (
        ind.astype(jnp.int32) + (jnp.arange(B, dtype=jnp.int32) * N)[:, None, None]
    ).reshape(_NUM_WORKERS, n_chunks, _CHUNK)

    x_words = _tc_pack(X.reshape(B * N, D), rows_per_block=1024)
    w16 = _tc_cast_w(W)
    xg_words = _sc_gather(x_words, idx)
    return _tc_matmul(xg_words.reshape(B, H, K, D // 2), w16)


# speedup vs baseline: 1.0086x; 1.0086x over previous
import functools

import jax
import jax.numpy as jnp
from jax import lax
from jax.experimental import pallas as pl
from jax.experimental.pallas import tpu as pltpu
from jax.experimental.pallas import tpu_sc as plsc

_NUM_CORES = 2
_NUM_SUBCORES = 16
_NUM_WORKERS = _NUM_CORES * _NUM_SUBCORES
_CHUNK = 32
_NBUF = 4


def _tc_pack(x, rows_per_block):
    R, D = x.shape
    Dh = D // 2

    def body(x_ref, out_ref):
        u = lax.bitcast_convert_type(x_ref[...], jnp.uint32)
        au = u[:, :Dh] + 0x8000
        bu = u[:, Dh:] + 0x8000
        out_ref[...] = lax.bitcast_convert_type(
            (au >> 16) | (bu & jnp.uint32(0xFFFF0000)), jnp.int32
        )

    return pl.pallas_call(
        body,
        grid=(R // rows_per_block,),
        in_specs=[pl.BlockSpec((rows_per_block, D), lambda i: (i, 0))],
        out_specs=pl.BlockSpec((rows_per_block, Dh), lambda i: (i, 0)),
        out_shape=jax.ShapeDtypeStruct((R, Dh), jnp.int32),
    )(x)


def _sc_gather(x_words, idx):
    V, Dw = x_words.shape
    _, n_chunks, _ = idx.shape
    rows_per_w = n_chunks * _CHUNK
    total_rows = _NUM_WORKERS * rows_per_w
    n_groups = n_chunks // _NBUF

    mesh = plsc.VectorSubcoreMesh(
        core_axis_name="c",
        subcore_axis_name="s",
        num_cores=_NUM_CORES,
        num_subcores=_NUM_SUBCORES,
    )

    @functools.partial(
        pl.kernel,
        mesh=mesh,
        out_type=jax.ShapeDtypeStruct((total_rows, Dw), jnp.int32),
        scratch_types=[
            pltpu.VMEM((n_chunks, _CHUNK), jnp.int32),
            pltpu.VMEM((_NBUF, _CHUNK, Dw), jnp.int32),
            pltpu.SemaphoreType.DMA,
            pltpu.SemaphoreType.DMA,
        ],
    )
    def gather_kernel(idx_hbm, x_hbm, out_hbm, idx_v, bufs, gsem, ssem):
        wid = lax.axis_index("s") * _NUM_CORES + lax.axis_index("c")
        base = wid * rows_per_w
        pltpu.sync_copy(idx_hbm.at[wid], idx_v)

        def gather(c, i):
            return pltpu.make_async_copy(x_hbm.at[idx_v.at[c]], bufs.at[i], gsem)

        def store(c, i):
            return pltpu.make_async_copy(
                bufs.at[i], out_hbm.at[pl.ds(base + c * _CHUNK, _CHUNK)], ssem
            )

        for i in range(_NBUF):
            gather(i, i).start()

        def body(g, _):
            c0 = _NBUF * g
            for i in range(_NBUF):
                gather(c0 + i, i).wait()
                store(c0 + i, i).start()
            for i in range(_NBUF):
                store(c0 + i, i).wait()

                @pl.when(c0 + i + _NBUF < n_chunks)
                def _start_next(c=c0 + i + _NBUF, i=i):
                    gather(c, i).start()

            return None

        lax.fori_loop(0, n_groups, body, None)

    return gather_kernel(idx, x_words)


def _tc_cast_w(w):
    H, D, HD = w.shape

    def body(w_ref, out_ref):
        out_ref[...] = w_ref[...].astype(jnp.bfloat16)

    return pl.pallas_call(
        body,
        grid=(H,),
        in_specs=[pl.BlockSpec((1, D, HD), lambda h: (h, 0, 0))],
        out_specs=pl.BlockSpec((1, D, HD), lambda h: (h, 0, 0)),
        out_shape=jax.ShapeDtypeStruct((H, D, HD), jnp.bfloat16),
    )(w)


def _tc_matmul(xg_words, w):
    B, H, K, Dw = xg_words.shape
    D = 2 * Dw
    HD = w.shape[2]

    HPB = 16

    def body(xg_ref, w_ref, out_ref):
        dn = (((1,), (0,)), ((), ()))
        for i in range(HPB):
            words = lax.bitcast_convert_type(xg_ref[0, i], jnp.uint32)
            lo = lax.bitcast_convert_type(words << 16, jnp.float32).astype(
                jnp.bfloat16
            )
            hi = lax.bitcast_convert_type(
                words & jnp.uint32(0xFFFF0000), jnp.float32
            ).astype(jnp.bfloat16)
            out_ref[0, i] = lax.dot_general(
                lo, w_ref[i, :Dw], dn, preferred_element_type=jnp.float32
            ) + lax.dot_general(
                hi, w_ref[i, Dw:], dn, preferred_element_type=jnp.float32
            )

    return pl.pallas_call(
        body,
        grid=(H // HPB, B),
        in_specs=[
            pl.BlockSpec((1, HPB, K, Dw), lambda h, b: (b, h, 0, 0)),
            pl.BlockSpec((HPB, D, HD), lambda h, b: (h, 0, 0)),
        ],
        out_specs=pl.BlockSpec((1, HPB, K, HD), lambda h, b: (b, h, 0, 0)),
        out_shape=jax.ShapeDtypeStruct((B, H, K, HD), jnp.float32),
    )(xg_words, w)


def kernel(X, ind, W):
    B, N, D = X.shape
    _, H, K = ind.shape

    total_rows = B * H * K
    rows_per_w = total_rows // _NUM_WORKERS
    n_chunks = rows_per_w // _CHUNK

    idx = (
        ind.astype(jnp.int32) + (jnp.arange(B, dtype=jnp.int32) * N)[:, None, None]
    ).reshape(_NUM_WORKERS, n_chunks, _CHUNK)

    x_words = _tc_pack(X.reshape(B * N, D), rows_per_block=2048)
    w16 = _tc_cast_w(W)
    xg_words = _sc_gather(x_words, idx)
    return _tc_matmul(xg_words.reshape(B, H, K, D // 2), w16)
